# Initial kernel scaffold; baseline (speedup 1.0000x reference)
import jax
import jax.numpy as jnp
from jax.experimental import pallas as pl


def kernel(x, table, W1, b1, W2, b2):
    raise NotImplementedError("write your pallas kernel here")


# speedup vs baseline: 4.3597x; 4.3597x over previous
import functools

import jax
import jax.numpy as jnp
from jax import lax
from jax.experimental import pallas as pl
from jax.experimental.pallas import tpu as pltpu
from jax.experimental.pallas import tpu_sc as plsc

BATCH = 16384
HIDDEN = 128
EMBED = 64


def _sc_gather(table, idx):
    info = plsc.get_sparse_core_info()
    nw = info.num_cores * info.num_subcores
    bpw = BATCH // nw
    mesh = plsc.VectorSubcoreMesh(core_axis_name="c", subcore_axis_name="s")

    @functools.partial(
        pl.kernel,
        mesh=mesh,
        out_type=jax.ShapeDtypeStruct((BATCH, HIDDEN), jnp.float32),
        scratch_types=[
            pltpu.VMEM((bpw,), jnp.int32),
            pltpu.VMEM((bpw, HIDDEN), jnp.float32),
            pltpu.SemaphoreType.DMA,
        ],
    )
    def k(table_hbm, idx_hbm, out_hbm, idx_v, rows_v, sem):
        wid = lax.axis_index("s") * info.num_cores + lax.axis_index("c")
        base = wid * bpw
        pltpu.sync_copy(idx_hbm.at[pl.ds(base, bpw)], idx_v)
        pltpu.async_copy(table_hbm.at[idx_v], rows_v, sem).wait()
        pltpu.sync_copy(rows_v, out_hbm.at[pl.ds(base, bpw)])

    return k(table, idx)


_BLK = 1024


def _mlp_body(g_ref, w1_ref, b1_ref, w2_ref, b2_ref, out_ref):
    g = g_ref[...]
    h = jnp.dot(g, w1_ref[...], preferred_element_type=jnp.float32)
    h = jnp.maximum(h + b1_ref[...], 0.0)
    o = jnp.dot(h, w2_ref[...], preferred_element_type=jnp.float32)
    o = o + b2_ref[...]
    n = jnp.sqrt(jnp.sum(o * o, axis=1, keepdims=True))
    out_ref[...] = o / jnp.maximum(n, 1e-12)


def _tc_mlp(g, W1, b1, W2, b2):
    return pl.pallas_call(
        _mlp_body,
        grid=(BATCH // _BLK,),
        in_specs=[
            pl.BlockSpec((_BLK, HIDDEN), lambda i: (i, 0)),
            pl.BlockSpec((HIDDEN, HIDDEN), lambda i: (0, 0)),
            pl.BlockSpec((1, HIDDEN), lambda i: (0, 0)),
            pl.BlockSpec((HIDDEN, EMBED), lambda i: (0, 0)),
            pl.BlockSpec((1, EMBED), lambda i: (0, 0)),
        ],
        out_specs=pl.BlockSpec((_BLK, EMBED), lambda i: (i, 0)),
        out_shape=jax.ShapeDtypeStruct((BATCH, EMBED), jnp.float32),
    )(g, W1, b1.reshape(1, HIDDEN), W2, b2.reshape(1, EMBED))


def kernel(x, table, W1, b1, W2, b2):
    g = _sc_gather(table, x)
    return _tc_mlp(g, W1, b1, W2, b2)


# speedup vs baseline: 5.3038x; 1.2165x over previous
import functools

import jax
import jax.numpy as jnp
from jax import lax
from jax.experimental import pallas as pl
from jax.experimental.pallas import tpu as pltpu
from jax.experimental.pallas import tpu_sc as plsc

BATCH = 16384
HIDDEN = 128
EMBED = 64


def _sc_gather(table, idx):
    info = plsc.get_sparse_core_info()
    nw = info.num_cores * info.num_subcores
    bpw = BATCH // nw
    mesh = plsc.VectorSubcoreMesh(core_axis_name="c", subcore_axis_name="s")

    @functools.partial(
        pl.kernel,
        mesh=mesh,
        out_type=jax.ShapeDtypeStruct((BATCH, HIDDEN), jnp.float32),
        scratch_types=[
            pltpu.VMEM((bpw,), jnp.int32),
            pltpu.VMEM((bpw, HIDDEN), jnp.float32),
            pltpu.SemaphoreType.DMA,
        ],
    )
    def k(table_hbm, idx_hbm, out_hbm, idx_v, rows_v, sem):
        wid = lax.axis_index("s") * info.num_cores + lax.axis_index("c")
        base = wid * bpw
        pltpu.sync_copy(idx_hbm.at[pl.ds(base, bpw)], idx_v)
        pltpu.async_copy(table_hbm.at[idx_v], rows_v, sem).wait()
        pltpu.sync_copy(rows_v, out_hbm.at[pl.ds(base, bpw)])

    return k(table, idx)


_BLK = 1024


def _mlp_body(g_ref, w1_ref, b1_ref, w2t_ref, b2_ref, out_ref):
    g = g_ref[...]
    h = jnp.dot(g, w1_ref[...], preferred_element_type=jnp.float32)
    h = jnp.maximum(h + b1_ref[...], 0.0)
    ot = lax.dot_general(w2t_ref[...], h, (((1,), (1,)), ((), ())),
                         preferred_element_type=jnp.float32)
    ot = ot + b2_ref[...]
    n2 = jnp.sum(ot * ot, axis=0, keepdims=True)
    out_ref[...] = ot * jnp.minimum(lax.rsqrt(n2), 1e12)


def _tc_mlp(g, W1, b1, W2t, b2):
    return pl.pallas_call(
        _mlp_body,
        grid=(BATCH // _BLK,),
        in_specs=[
            pl.BlockSpec((_BLK, HIDDEN), lambda i: (i, 0)),
            pl.BlockSpec((HIDDEN, HIDDEN), lambda i: (0, 0)),
            pl.BlockSpec((1, HIDDEN), lambda i: (0, 0)),
            pl.BlockSpec((EMBED, HIDDEN), lambda i: (0, 0)),
            pl.BlockSpec((EMBED, 1), lambda i: (0, 0)),
        ],
        out_specs=pl.BlockSpec((EMBED, _BLK), lambda i: (0, i)),
        out_shape=jax.ShapeDtypeStruct((EMBED, BATCH), jnp.float32),
    )(g, W1, b1.reshape(1, HIDDEN), W2t, b2.reshape(EMBED, 1))


def kernel(x, table, W1, b1, W2, b2):
    g = _sc_gather(table, x)
    out_t = _tc_mlp(g, W1, b1, W2.T, b2)
    return out_t.T


# speedup vs baseline: 6.4159x; 1.2097x over previous
import functools

import jax
import jax.numpy as jnp
from jax import lax
from jax.experimental import pallas as pl
from jax.experimental.pallas import tpu as pltpu
from jax.experimental.pallas import tpu_sc as plsc

BATCH = 16384
HIDDEN = 128
EMBED = 64


def _sc_gather(table, idx):
    info = plsc.get_sparse_core_info()
    nw = info.num_cores * info.num_subcores
    bpw = BATCH // nw
    mesh = plsc.VectorSubcoreMesh(core_axis_name="c", subcore_axis_name="s")

    @functools.partial(
        pl.kernel,
        mesh=mesh,
        out_type=jax.ShapeDtypeStruct((BATCH, HIDDEN), jnp.float32),
        scratch_types=[
            pltpu.VMEM((bpw,), jnp.int32),
            pltpu.VMEM((bpw, HIDDEN), jnp.float32),
            pltpu.SemaphoreType.DMA,
        ],
    )
    def k(table_hbm, idx_hbm, out_hbm, idx_v, rows_v, sem):
        wid = lax.axis_index("s") * info.num_cores + lax.axis_index("c")
        base = wid * bpw
        pltpu.sync_copy(idx_hbm.at[pl.ds(base, bpw)], idx_v)
        pltpu.async_copy(table_hbm.at[idx_v], rows_v, sem).wait()
        pltpu.sync_copy(rows_v, out_hbm.at[pl.ds(base, bpw)])

    return k(table, idx)


_BLK = 4096


def _mlp_body(g_ref, w1_ref, b1_ref, w2t_ref, b2_ref, out_ref):
    g = g_ref[...]
    h = jnp.dot(g, w1_ref[...], preferred_element_type=jnp.float32)
    h = jnp.maximum(h + b1_ref[...], 0.0)
    ot = lax.dot_general(w2t_ref[...], h, (((1,), (1,)), ((), ())),
                         preferred_element_type=jnp.float32)
    ot = ot + b2_ref[...]
    n2 = jnp.sum(ot * ot, axis=0, keepdims=True)
    out_ref[...] = ot * jnp.minimum(lax.rsqrt(n2), 1e12)


def _tc_mlp(g, W1, b1, W2t, b2):
    return pl.pallas_call(
        _mlp_body,
        grid=(BATCH // _BLK,),
        in_specs=[
            pl.BlockSpec((_BLK, HIDDEN), lambda i: (i, 0)),
            pl.BlockSpec((HIDDEN, HIDDEN), lambda i: (0, 0)),
            pl.BlockSpec((1, HIDDEN), lambda i: (0, 0)),
            pl.BlockSpec((EMBED, HIDDEN), lambda i: (0, 0)),
            pl.BlockSpec((EMBED, 1), lambda i: (0, 0)),
        ],
        out_specs=pl.BlockSpec((EMBED, _BLK), lambda i: (0, i)),
        out_shape=jax.ShapeDtypeStruct((EMBED, BATCH), jnp.float32),
    )(g, W1, b1.reshape(1, HIDDEN), W2t, b2.reshape(EMBED, 1))


def kernel(x, table, W1, b1, W2, b2):
    g = _sc_gather(table, x)
    out_t = _tc_mlp(g, W1, b1, W2.T, b2)
    return out_t.T


# speedup vs baseline: 6.6033x; 1.0292x over previous
import functools

import jax
import jax.numpy as jnp
from jax import lax
from jax.experimental import pallas as pl
from jax.experimental.pallas import tpu as pltpu
from jax.experimental.pallas import tpu_sc as plsc

BATCH = 16384
HIDDEN = 128
EMBED = 64


def _sc_gather(table, idx):
    info = plsc.get_sparse_core_info()
    nw = info.num_cores * info.num_subcores
    bpw = BATCH // nw
    mesh = plsc.VectorSubcoreMesh(core_axis_name="c", subcore_axis_name="s")

    @functools.partial(
        pl.kernel,
        mesh=mesh,
        out_type=jax.ShapeDtypeStruct((BATCH, HIDDEN), jnp.float32),
        scratch_types=[
            pltpu.VMEM((bpw,), jnp.int32),
            pltpu.VMEM((bpw, HIDDEN), jnp.float32),
            pltpu.SemaphoreType.DMA,
        ],
    )
    def k(table_hbm, idx_hbm, out_hbm, idx_v, rows_v, sem):
        wid = lax.axis_index("s") * info.num_cores + lax.axis_index("c")
        base = wid * bpw
        pltpu.sync_copy(idx_hbm.at[pl.ds(base, bpw)], idx_v)
        pltpu.async_copy(table_hbm.at[idx_v], rows_v, sem).wait()
        pltpu.sync_copy(rows_v, out_hbm.at[pl.ds(base, bpw)])

    return k(table, idx)


_BLK = 8192


def _mlp_body(g_ref, w1_ref, b1_ref, w2t_ref, b2_ref, out_ref):
    g = g_ref[...]
    h = jnp.dot(g, w1_ref[...], preferred_element_type=jnp.float32)
    h = jnp.maximum(h + b1_ref[...], 0.0)
    ot = lax.dot_general(w2t_ref[...], h, (((1,), (1,)), ((), ())),
                         preferred_element_type=jnp.float32)
    ot = ot + b2_ref[...]
    n2 = jnp.sum(ot * ot, axis=0, keepdims=True)
    out_ref[...] = ot * jnp.minimum(lax.rsqrt(n2), 1e12)


def _tc_mlp(g, W1, b1, W2t, b2):
    return pl.pallas_call(
        _mlp_body,
        grid=(BATCH // _BLK,),
        in_specs=[
            pl.BlockSpec((_BLK, HIDDEN), lambda i: (i, 0)),
            pl.BlockSpec((HIDDEN, HIDDEN), lambda i: (0, 0)),
            pl.BlockSpec((1, HIDDEN), lambda i: (0, 0)),
            pl.BlockSpec((EMBED, HIDDEN), lambda i: (0, 0)),
            pl.BlockSpec((EMBED, 1), lambda i: (0, 0)),
        ],
        out_specs=pl.BlockSpec((EMBED, _BLK), lambda i: (0, i)),
        out_shape=jax.ShapeDtypeStruct((EMBED, BATCH), jnp.float32),
    )(g, W1, b1.reshape(1, HIDDEN), W2t, b2.reshape(EMBED, 1))


def kernel(x, table, W1, b1, W2, b2):
    g = _sc_gather(table, x)
    out_t = _tc_mlp(g, W1, b1, W2.T, b2)
    return out_t.T


# speedup vs baseline: 6.6204x; 1.0026x over previous
import functools

import jax
import jax.numpy as jnp
from jax import lax
from jax.experimental import pallas as pl
from jax.experimental.pallas import tpu as pltpu
from jax.experimental.pallas import tpu_sc as plsc

BATCH = 16384
HIDDEN = 128
EMBED = 64


def _sc_gather(table, idx):
    info = plsc.get_sparse_core_info()
    nw = info.num_cores * info.num_subcores
    bpw = BATCH // nw
    mesh = plsc.VectorSubcoreMesh(core_axis_name="c", subcore_axis_name="s")

    @functools.partial(
        pl.kernel,
        mesh=mesh,
        out_type=jax.ShapeDtypeStruct((BATCH, HIDDEN), jnp.float32),
        scratch_types=[
            pltpu.VMEM((bpw,), jnp.int32),
            pltpu.VMEM((bpw, HIDDEN), jnp.float32),
            pltpu.SemaphoreType.DMA,
        ],
    )
    def k(table_hbm, idx_hbm, out_hbm, idx_v, rows_v, sem):
        wid = lax.axis_index("s") * info.num_cores + lax.axis_index("c")
        base = wid * bpw
        pltpu.sync_copy(idx_hbm.at[pl.ds(base, bpw)], idx_v)
        pltpu.async_copy(table_hbm.at[idx_v], rows_v, sem).wait()
        pltpu.sync_copy(rows_v, out_hbm.at[pl.ds(base, bpw)])

    return k(table, idx)


_BLK = 8192


def _mlp_body(g_ref, w1_ref, b1_ref, w2t_ref, b2_ref, out_ref):
    g = g_ref[...].astype(jnp.bfloat16)
    h = jnp.dot(g, w1_ref[...].astype(jnp.bfloat16),
                preferred_element_type=jnp.float32)
    h = jnp.maximum(h + b1_ref[...], 0.0).astype(jnp.bfloat16)
    ot = lax.dot_general(w2t_ref[...].astype(jnp.bfloat16), h,
                         (((1,), (1,)), ((), ())),
                         preferred_element_type=jnp.float32)
    ot = ot + b2_ref[...]
    n2 = jnp.sum(ot * ot, axis=0, keepdims=True)
    out_ref[...] = ot * jnp.minimum(lax.rsqrt(n2), 1e12)


def _tc_mlp(g, W1, b1, W2t, b2):
    return pl.pallas_call(
        _mlp_body,
        grid=(BATCH // _BLK,),
        in_specs=[
            pl.BlockSpec((_BLK, HIDDEN), lambda i: (i, 0)),
            pl.BlockSpec((HIDDEN, HIDDEN), lambda i: (0, 0)),
            pl.BlockSpec((1, HIDDEN), lambda i: (0, 0)),
            pl.BlockSpec((EMBED, HIDDEN), lambda i: (0, 0)),
            pl.BlockSpec((EMBED, 1), lambda i: (0, 0)),
        ],
        out_specs=pl.BlockSpec((EMBED, _BLK), lambda i: (0, i)),
        out_shape=jax.ShapeDtypeStruct((EMBED, BATCH), jnp.float32),
    )(g, W1, b1.reshape(1, HIDDEN), W2t, b2.reshape(EMBED, 1))


def kernel(x, table, W1, b1, W2, b2):
    g = _sc_gather(table, x)
    out_t = _tc_mlp(g, W1, b1, W2.T, b2)
    return out_t.T


# speedup vs baseline: 6.6277x; 1.0011x over previous
import functools

import jax
import jax.numpy as jnp
from jax import lax
from jax.experimental import pallas as pl
from jax.experimental.pallas import tpu as pltpu
from jax.experimental.pallas import tpu_sc as plsc

BATCH = 16384
HIDDEN = 128
EMBED = 64


def _sc_gather(table, idx):
    info = plsc.get_sparse_core_info()
    nw = info.num_cores * info.num_subcores
    bpw = BATCH // nw
    mesh = plsc.VectorSubcoreMesh(core_axis_name="c", subcore_axis_name="s")

    @functools.partial(
        pl.kernel,
        mesh=mesh,
        out_type=jax.ShapeDtypeStruct((BATCH, HIDDEN), jnp.float32),
        scratch_types=[
            pltpu.VMEM((bpw,), jnp.int32),
            pltpu.VMEM((bpw, HIDDEN), jnp.float32),
            pltpu.SemaphoreType.DMA,
        ],
    )
    def k(table_hbm, idx_hbm, out_hbm, idx_v, rows_v, sem):
        wid = lax.axis_index("s") * info.num_cores + lax.axis_index("c")
        base = wid * bpw
        pltpu.sync_copy(idx_hbm.at[pl.ds(base, bpw)], idx_v)
        pltpu.async_copy(table_hbm.at[idx_v], rows_v, sem).wait()
        pltpu.sync_copy(rows_v, out_hbm.at[pl.ds(base, bpw)])

    return k(table, idx)


_BLK = 8192


def _mlp_body(g_ref, w1_ref, b1_ref, w2t_ref, b2_ref, out_ref):
    g = g_ref[...]
    h = jnp.dot(g, w1_ref[...], preferred_element_type=jnp.float32)
    h = jnp.maximum(h + b1_ref[...], 0.0)
    ot = lax.dot_general(w2t_ref[...], h, (((1,), (1,)), ((), ())),
                         preferred_element_type=jnp.float32)
    ot = ot + b2_ref[...]
    n2 = jnp.sum(ot * ot, axis=0, keepdims=True)
    out_ref[...] = ot * jnp.minimum(lax.rsqrt(n2), 1e12)


def _tc_mlp(g, W1, b1, W2t, b2):
    return pl.pallas_call(
        _mlp_body,
        grid=(BATCH // _BLK,),
        in_specs=[
            pl.BlockSpec((_BLK, HIDDEN), lambda i: (i, 0)),
            pl.BlockSpec((HIDDEN, HIDDEN), lambda i: (0, 0)),
            pl.BlockSpec((1, HIDDEN), lambda i: (0, 0)),
            pl.BlockSpec((EMBED, HIDDEN), lambda i: (0, 0)),
            pl.BlockSpec((EMBED, 1), lambda i: (0, 0)),
        ],
        out_specs=pl.BlockSpec((EMBED, _BLK), lambda i: (0, i)),
        out_shape=jax.ShapeDtypeStruct((EMBED, BATCH), jnp.float32),
    )(g, W1, b1.reshape(1, HIDDEN), W2t, b2.reshape(EMBED, 1))


def kernel(x, table, W1, b1, W2, b2):
    g = _sc_gather(table, x)
    out_t = _tc_mlp(g, W1, b1, W2.T, b2)
    return out_t.T
